# baseline (speedup 1.0000x reference)
import functools

import jax
import jax.numpy as jnp
from jax import lax
from jax.experimental import pallas as pl
from jax.experimental.pallas import tpu as pltpu
from jax.experimental.pallas import tpu_sc as plsc

N = 10000
E = 320000
D = 128
W = 64
A = 40000
S = 5000

NC = 2
NT = 16
CH = 128

NPAD = 10112
SPAD = 5120

ECHUNKS_PER_TILE = 80
ACHUNKS_PER_TILE = 10

BM = 632


def _elu(v):
    return jnp.where(v > 0, v, jnp.exp(jnp.minimum(v, 0.0)) - 1.0)


def _make_sc_segment_sum(npad, chunks_per_tile):
    mesh = plsc.VectorSubcoreMesh(core_axis_name="c", subcore_axis_name="s")
    zr = npad // NT
    nb = chunks_per_tile // 2

    @functools.partial(
        pl.kernel,
        out_type=jax.ShapeDtypeStruct((NC * npad, W), jnp.float32),
        mesh=mesh,
        scratch_types=[
            pltpu.VMEM((chunks_per_tile, 2, CH), jnp.int32),
            pltpu.VMEM((CH, W), jnp.float32),
            pltpu.VMEM((CH, W), jnp.float32),
            pltpu.SemaphoreType.DMA,
            pltpu.SemaphoreType.DMA,
            pltpu.SemaphoreType.DMA,
            pltpu.SemaphoreType.DMA,
            pltpu.SemaphoreType.DMA,
            pltpu.VMEM_SHARED((npad, W), jnp.float32),
        ],
        compiler_params=pltpu.CompilerParams(use_tc_tiling_on_sc=False),
    )
    def seg(table_hbm, idx_hbm, out_hbm, idxall, rows0, rows1,
            sem_i, sem_g0, sem_g1, sem_s0, sem_s1, acc):
        cid = lax.axis_index("c")
        sid = lax.axis_index("s")
        c0 = (cid * NT + sid) * chunks_per_tile

        idx_cp = pltpu.async_copy(idx_hbm.at[pl.ds(c0, chunks_per_tile)],
                                  idxall, sem_i)
        tbl = table_hbm
        z16 = jnp.zeros((16,), jnp.float32)

        def zbody(i, carry):
            for k in range(W // 16):
                rows0[i, pl.ds(k * 16, 16)] = z16
            return carry

        lax.fori_loop(0, CH, zbody, 0)
        zbase = pl.multiple_of(sid * zr, 8)
        nfull, rem = zr // CH, zr % CH
        for q in range(nfull):
            pltpu.sync_copy(rows0, acc.at[pl.ds(zbase + q * CH, CH)])
        if rem:
            pltpu.sync_copy(rows0.at[pl.ds(0, rem)],
                            acc.at[pl.ds(zbase + nfull * CH, rem)])
        idx_cp.wait()
        plsc.subcore_barrier()
        pltpu.async_copy(tbl.at[idxall.at[0, 0]], rows0, sem_g0)

        def body(k, carry):
            j0 = 2 * k
            j1 = j0 + 1
            pltpu.make_async_copy(tbl.at[idxall.at[j0, 0]],
                                  rows0, sem_g0).wait()
            pltpu.async_copy(rows0, acc.at[idxall.at[j0, 1]], sem_s0, add=True)

            @pl.when(k > 0)
            def _():
                pltpu.make_async_copy(rows1, acc.at[idxall.at[j1, 1]],
                                      sem_s1).wait()

            pltpu.async_copy(tbl.at[idxall.at[j1, 0]], rows1, sem_g1)
            pltpu.make_async_copy(tbl.at[idxall.at[j1, 0]],
                                  rows1, sem_g1).wait()
            pltpu.async_copy(rows1, acc.at[idxall.at[j1, 1]], sem_s1, add=True)
            pltpu.make_async_copy(rows0, acc.at[idxall.at[j0, 1]],
                                  sem_s0).wait()

            @pl.when(k < nb - 1)
            def _():
                pltpu.async_copy(tbl.at[idxall.at[j0 + 2, 0]],
                                 rows0, sem_g0)

            return carry

        lax.fori_loop(0, nb, body, 0)
        pltpu.make_async_copy(rows1, acc.at[idxall.at[0, 1]], sem_s1).wait()
        plsc.subcore_barrier()

        pltpu.sync_copy(acc.at[pl.ds(zbase, zr)],
                        out_hbm.at[pl.ds(pl.multiple_of(cid * npad + zbase, 8), zr)])

    return seg


_seg_edge = _make_sc_segment_sum(NPAD, ECHUNKS_PER_TILE)
_seg_pool = _make_sc_segment_sum(SPAD, ACHUNKS_PER_TILE)


def _pack_idx(gidx, sidx, total, dump_row):
    n = gidx.shape[0]
    pad = total - n
    g = jnp.concatenate([gidx.astype(jnp.int32),
                         jnp.zeros((pad,), jnp.int32)])
    s = jnp.concatenate([sidx.astype(jnp.int32),
                         jnp.full((pad,), dump_row, jnp.int32)])
    return jnp.stack([g.reshape(-1, CH), s.reshape(-1, CH)], axis=1)


def _layer0_tc(x, alo0, alo1, ahi0, ahi1, w_root, w_rel, b):
    def body(x_ref, lo0, lo1, hi0, hi1, wr_ref, wl_ref, b_ref, o_ref):
        agg = jnp.concatenate([lo0[...] + lo1[...], hi0[...] + hi1[...]],
                              axis=1)
        o_ref[...] = _elu(
            jnp.dot(x_ref[...], wr_ref[...], preferred_element_type=jnp.float32)
            + jnp.dot(agg, wl_ref[...], preferred_element_type=jnp.float32)
            + b_ref[...])

    return pl.pallas_call(
        body,
        grid=(NPAD // BM,),
        in_specs=[
            pl.BlockSpec((BM, D), lambda i: (i, 0)),
            pl.BlockSpec((BM, W), lambda i: (i, 0)),
            pl.BlockSpec((BM, W), lambda i: (i, 0)),
            pl.BlockSpec((BM, W), lambda i: (i, 0)),
            pl.BlockSpec((BM, W), lambda i: (i, 0)),
            pl.BlockSpec((D, W), lambda i: (0, 0)),
            pl.BlockSpec((D, W), lambda i: (0, 0)),
            pl.BlockSpec((1, W), lambda i: (0, 0)),
        ],
        out_specs=pl.BlockSpec((BM, W), lambda i: (i, 0)),
        out_shape=jax.ShapeDtypeStruct((NPAD, W), jnp.float32),
    )(x, alo0, alo1, ahi0, ahi1, w_root, w_rel, b.reshape(1, W))


def _layer_tc(h, agg0, agg1, w_root, w_rel, b):
    def body(h_ref, a0_ref, a1_ref, wr_ref, wl_ref, b_ref, o_ref):
        agg = a0_ref[...] + a1_ref[...]
        o_ref[...] = _elu(
            jnp.dot(h_ref[...], wr_ref[...], preferred_element_type=jnp.float32)
            + jnp.dot(agg, wl_ref[...], preferred_element_type=jnp.float32)
            + b_ref[...])

    return pl.pallas_call(
        body,
        grid=(NPAD // BM,),
        in_specs=[
            pl.BlockSpec((BM, W), lambda i: (i, 0)),
            pl.BlockSpec((BM, W), lambda i: (i, 0)),
            pl.BlockSpec((BM, W), lambda i: (i, 0)),
            pl.BlockSpec((W, W), lambda i: (0, 0)),
            pl.BlockSpec((W, W), lambda i: (0, 0)),
            pl.BlockSpec((1, W), lambda i: (0, 0)),
        ],
        out_specs=pl.BlockSpec((BM, W), lambda i: (i, 0)),
        out_shape=jax.ShapeDtypeStruct((NPAD, W), jnp.float32),
    )(h, agg0, agg1, w_root, w_rel, b.reshape(1, W))


def _head(p0, p1, fc1_w, fc1_b, fc2_w, fc2_b, fc3_w, fc3_b):
    SB = 1000

    def body(p0_ref, p1_ref, w1_ref, b1_ref, w2_ref, b2_ref, w3_ref, b3_ref, o_ref):
        p = p0_ref[...] + p1_ref[...]
        h = _elu(jnp.dot(p, w1_ref[...], preferred_element_type=jnp.float32) + b1_ref[...])
        h = _elu(jnp.dot(h, w2_ref[...], preferred_element_type=jnp.float32) + b2_ref[...])
        lg = jnp.dot(h, w3_ref[...], preferred_element_type=jnp.float32) + b3_ref[...]
        m = jnp.max(lg, axis=1, keepdims=True)
        e = jnp.exp(lg - m)
        o_ref[...] = (lg - m) - jnp.log(jnp.sum(e, axis=1, keepdims=True))

    return pl.pallas_call(
        body,
        grid=(S // SB,),
        in_specs=[
            pl.BlockSpec((SB, W), lambda i: (i, 0)),
            pl.BlockSpec((SB, W), lambda i: (i, 0)),
            pl.BlockSpec((W, W), lambda i: (0, 0)),
            pl.BlockSpec((1, W), lambda i: (0, 0)),
            pl.BlockSpec((W, 32), lambda i: (0, 0)),
            pl.BlockSpec((1, 32), lambda i: (0, 0)),
            pl.BlockSpec((32, 2), lambda i: (0, 0)),
            pl.BlockSpec((1, 2), lambda i: (0, 0)),
        ],
        out_specs=pl.BlockSpec((SB, 2), lambda i: (i, 0)),
        out_shape=jax.ShapeDtypeStruct((S, 2), jnp.float32),
    )(p0, p1, fc1_w, fc1_b.reshape(1, W), fc2_w, fc2_b.reshape(1, 32),
      fc3_w, fc3_b.reshape(1, 2))


def kernel(x, edge_index, assign_nodes, assign_set_ids,
           W0_root, W0_rel, b0,
           W1_root, W1_rel, b1,
           W2_root, W2_rel, b2,
           fc1_w, fc1_b, fc2_w, fc2_b, fc3_w, fc3_b):
    src = edge_index[0].astype(jnp.int32)
    dst = edge_index[1].astype(jnp.int32)
    eidx = _pack_idx(src, dst, NC * NT * ECHUNKS_PER_TILE * CH, NPAD - 1)
    aidx = _pack_idx(assign_set_ids, assign_nodes,
                     NC * NT * ACHUNKS_PER_TILE * CH, SPAD - 1)

    xp = jnp.concatenate([x, jnp.zeros((NPAD - N, D), jnp.float32)])
    alo = _seg_edge(xp[:, :W], eidx)
    ahi = _seg_edge(xp[:, W:], eidx)
    h1 = _layer0_tc(xp, alo[:NPAD], alo[NPAD:], ahi[:NPAD], ahi[NPAD:],
                    W0_root, W0_rel, b0)
    agg = _seg_edge(h1, eidx)
    h2 = _layer_tc(h1, agg[:NPAD], agg[NPAD:], W1_root, W1_rel, b1)
    agg = _seg_edge(h2, eidx)
    h3 = _layer_tc(h2, agg[:NPAD], agg[NPAD:], W2_root, W2_rel, b2)

    pooled = _seg_pool(h3, aidx)
    return _head(pooled[:S], pooled[SPAD:SPAD + S],
                 fc1_w, fc1_b, fc2_w, fc2_b, fc3_w, fc3_b)


# speedup vs baseline: 2.5220x; 2.5220x over previous
import functools

import jax
import jax.numpy as jnp
from jax import lax
from jax.experimental import pallas as pl
from jax.experimental.pallas import tpu as pltpu
from jax.experimental.pallas import tpu_sc as plsc

N = 10000
E = 320000
D = 128
W = 64
A = 40000
S = 5000

NC = 2
NT = 16
CH = 128

NPAD = 10112
SPAD = 5120

ECHUNKS_PER_TILE = 80
ACHUNKS_PER_TILE = 10

BM = 632


def _elu(v):
    return jnp.where(v > 0, v, jnp.exp(jnp.minimum(v, 0.0)) - 1.0)


def _make_sc_segment_sum(npad, chunks_per_tile):
    mesh = plsc.VectorSubcoreMesh(core_axis_name="c", subcore_axis_name="s")
    zr = npad // NT
    tr = NPAD // NT
    nb = chunks_per_tile // 2

    @functools.partial(
        pl.kernel,
        out_type=jax.ShapeDtypeStruct((NC * npad, W), jnp.float32),
        mesh=mesh,
        scratch_types=[
            pltpu.VMEM((chunks_per_tile, 2, CH), jnp.int32),
            pltpu.VMEM((CH, W), jnp.float32),
            pltpu.VMEM((CH, W), jnp.float32),
            pltpu.SemaphoreType.DMA,
            pltpu.SemaphoreType.DMA,
            pltpu.SemaphoreType.DMA,
            pltpu.SemaphoreType.DMA,
            pltpu.SemaphoreType.DMA,
            pltpu.SemaphoreType.DMA,
            pltpu.VMEM_SHARED((NPAD, W), jnp.float32),
            pltpu.VMEM_SHARED((npad, W), jnp.float32),
        ],
        compiler_params=pltpu.CompilerParams(use_tc_tiling_on_sc=False),
    )
    def seg(table_hbm, idx_hbm, out_hbm, idxall, rows0, rows1,
            sem_i, sem_t, sem_g0, sem_g1, sem_s0, sem_s1, tbl, acc):
        cid = lax.axis_index("c")
        sid = lax.axis_index("s")
        c0 = (cid * NT + sid) * chunks_per_tile

        idx_cp = pltpu.async_copy(idx_hbm.at[pl.ds(c0, chunks_per_tile)],
                                  idxall, sem_i)
        tbase = pl.multiple_of(sid * tr, 8)
        tbl_cp = pltpu.async_copy(table_hbm.at[pl.ds(tbase, tr)],
                                  tbl.at[pl.ds(tbase, tr)], sem_t)
        z16 = jnp.zeros((16,), jnp.float32)

        def zbody(i, carry):
            for k in range(W // 16):
                rows0[i, pl.ds(k * 16, 16)] = z16
            return carry

        lax.fori_loop(0, CH, zbody, 0)
        zbase = pl.multiple_of(sid * zr, 8)
        nfull, rem = zr // CH, zr % CH
        for q in range(nfull):
            pltpu.sync_copy(rows0, acc.at[pl.ds(zbase + q * CH, CH)])
        if rem:
            pltpu.sync_copy(rows0.at[pl.ds(0, rem)],
                            acc.at[pl.ds(zbase + nfull * CH, rem)])
        idx_cp.wait()
        tbl_cp.wait()
        plsc.subcore_barrier()
        pltpu.async_copy(tbl.at[idxall.at[0, 0]], rows0, sem_g0)

        def body(k, carry):
            j0 = 2 * k
            j1 = j0 + 1
            pltpu.make_async_copy(tbl.at[idxall.at[j0, 0]],
                                  rows0, sem_g0).wait()
            pltpu.async_copy(rows0, acc.at[idxall.at[j0, 1]], sem_s0, add=True)

            @pl.when(k > 0)
            def _():
                pltpu.make_async_copy(rows1, acc.at[idxall.at[j1, 1]],
                                      sem_s1).wait()

            pltpu.async_copy(tbl.at[idxall.at[j1, 0]], rows1, sem_g1)
            pltpu.make_async_copy(tbl.at[idxall.at[j1, 0]],
                                  rows1, sem_g1).wait()
            pltpu.async_copy(rows1, acc.at[idxall.at[j1, 1]], sem_s1, add=True)
            pltpu.make_async_copy(rows0, acc.at[idxall.at[j0, 1]],
                                  sem_s0).wait()

            @pl.when(k < nb - 1)
            def _():
                pltpu.async_copy(tbl.at[idxall.at[j0 + 2, 0]],
                                 rows0, sem_g0)

            return carry

        lax.fori_loop(0, nb, body, 0)
        pltpu.make_async_copy(rows1, acc.at[idxall.at[0, 1]], sem_s1).wait()
        plsc.subcore_barrier()

        pltpu.sync_copy(acc.at[pl.ds(zbase, zr)],
                        out_hbm.at[pl.ds(pl.multiple_of(cid * npad + zbase, 8), zr)])

    return seg


_seg_edge = _make_sc_segment_sum(NPAD, ECHUNKS_PER_TILE)
_seg_pool = _make_sc_segment_sum(SPAD, ACHUNKS_PER_TILE)


def _make_sc_l0():
    mesh = plsc.VectorSubcoreMesh(core_axis_name="c", subcore_axis_name="s")
    cpt = NC * ECHUNKS_PER_TILE
    hcpt = cpt // 2
    zr = NPAD // NT
    nb = hcpt // 2

    @functools.partial(
        pl.kernel,
        out_type=jax.ShapeDtypeStruct((NC * NPAD, W), jnp.float32),
        mesh=mesh,
        scratch_types=[
            pltpu.VMEM((hcpt, 2, CH), jnp.int32),
            pltpu.VMEM((CH, W), jnp.float32),
            pltpu.VMEM((CH, W), jnp.float32),
            pltpu.SemaphoreType.DMA,
            pltpu.SemaphoreType.DMA,
            pltpu.SemaphoreType.DMA,
            pltpu.SemaphoreType.DMA,
            pltpu.SemaphoreType.DMA,
            pltpu.SemaphoreType.DMA,
            pltpu.VMEM_SHARED((NPAD, W), jnp.float32),
            pltpu.VMEM_SHARED((NPAD, W), jnp.float32),
        ],
        compiler_params=pltpu.CompilerParams(use_tc_tiling_on_sc=False),
    )
    def seg(x2_hbm, idx_hbm, out_hbm, idxall, rows0, rows1,
            sem_i, sem_t, sem_g0, sem_g1, sem_s0, sem_s1, tbl, acc):
        cid = lax.axis_index("c")
        sid = lax.axis_index("s")
        c0 = sid * cpt

        idx_cp = pltpu.async_copy(idx_hbm.at[pl.ds(c0, hcpt)], idxall, sem_i)
        tbase = pl.multiple_of(sid * zr, 8)
        tbl_cp = pltpu.async_copy(
            x2_hbm.at[pl.ds(pl.multiple_of(cid * NPAD + tbase, 8), zr)],
            tbl.at[pl.ds(tbase, zr)], sem_t)
        z16 = jnp.zeros((16,), jnp.float32)

        def zbody(i, carry):
            for k in range(W // 16):
                rows0[i, pl.ds(k * 16, 16)] = z16
            return carry

        lax.fori_loop(0, CH, zbody, 0)
        zbase = pl.multiple_of(sid * zr, 8)
        nfull, rem = zr // CH, zr % CH
        for q in range(nfull):
            pltpu.sync_copy(rows0, acc.at[pl.ds(zbase + q * CH, CH)])
        if rem:
            pltpu.sync_copy(rows0.at[pl.ds(0, rem)],
                            acc.at[pl.ds(zbase + nfull * CH, rem)])
        idx_cp.wait()
        tbl_cp.wait()
        plsc.subcore_barrier()

        def body(k, carry):
            j0 = 2 * k
            j1 = j0 + 1
            pltpu.make_async_copy(tbl.at[idxall.at[j0, 0]],
                                  rows0, sem_g0).wait()
            pltpu.async_copy(rows0, acc.at[idxall.at[j0, 1]], sem_s0, add=True)

            @pl.when(k > 0)
            def _():
                pltpu.make_async_copy(rows1, acc.at[idxall.at[j1, 1]],
                                      sem_s1).wait()

            pltpu.async_copy(tbl.at[idxall.at[j1, 0]], rows1, sem_g1)
            pltpu.make_async_copy(tbl.at[idxall.at[j1, 0]],
                                  rows1, sem_g1).wait()
            pltpu.async_copy(rows1, acc.at[idxall.at[j1, 1]], sem_s1, add=True)
            pltpu.make_async_copy(rows0, acc.at[idxall.at[j0, 1]],
                                  sem_s0).wait()

            @pl.when(k < nb - 1)
            def _():
                pltpu.async_copy(tbl.at[idxall.at[j0 + 2, 0]],
                                 rows0, sem_g0)

            return carry

        pltpu.async_copy(tbl.at[idxall.at[0, 0]], rows0, sem_g0)
        lax.fori_loop(0, nb, body, 0)
        pltpu.make_async_copy(rows1, acc.at[idxall.at[0, 1]], sem_s1).wait()

        pltpu.sync_copy(idx_hbm.at[pl.ds(c0 + hcpt, hcpt)], idxall)
        pltpu.async_copy(tbl.at[idxall.at[0, 0]], rows0, sem_g0)
        lax.fori_loop(0, nb, body, 0)
        pltpu.make_async_copy(rows1, acc.at[idxall.at[0, 1]], sem_s1).wait()

        plsc.subcore_barrier()
        pltpu.sync_copy(acc.at[pl.ds(zbase, zr)],
                        out_hbm.at[pl.ds(pl.multiple_of(cid * NPAD + zbase, 8), zr)])

    return seg


_seg_l0 = _make_sc_l0()


def _pack_idx(gidx, sidx, total, dump_row):
    n = gidx.shape[0]
    pad = total - n
    g = jnp.concatenate([gidx.astype(jnp.int32),
                         jnp.zeros((pad,), jnp.int32)])
    s = jnp.concatenate([sidx.astype(jnp.int32),
                         jnp.full((pad,), dump_row, jnp.int32)])
    return jnp.stack([g.reshape(-1, CH), s.reshape(-1, CH)], axis=1)


def _layer0_tc(x, alo, ahi, w_root, w_rel, b):
    def body(x_ref, lo, hi, wr_ref, wl_ref, b_ref, o_ref):
        agg = jnp.concatenate([lo[...], hi[...]], axis=1)
        o_ref[...] = _elu(
            jnp.dot(x_ref[...], wr_ref[...], preferred_element_type=jnp.float32)
            + jnp.dot(agg, wl_ref[...], preferred_element_type=jnp.float32)
            + b_ref[...])

    return pl.pallas_call(
        body,
        grid=(NPAD // BM,),
        in_specs=[
            pl.BlockSpec((BM, D), lambda i: (i, 0)),
            pl.BlockSpec((BM, W), lambda i: (i, 0)),
            pl.BlockSpec((BM, W), lambda i: (i, 0)),
            pl.BlockSpec((D, W), lambda i: (0, 0)),
            pl.BlockSpec((D, W), lambda i: (0, 0)),
            pl.BlockSpec((1, W), lambda i: (0, 0)),
        ],
        out_specs=pl.BlockSpec((BM, W), lambda i: (i, 0)),
        out_shape=jax.ShapeDtypeStruct((NPAD, W), jnp.float32),
    )(x, alo, ahi, w_root, w_rel, b.reshape(1, W))


def _layer_tc(h, agg0, agg1, w_root, w_rel, b):
    def body(h_ref, a0_ref, a1_ref, wr_ref, wl_ref, b_ref, o_ref):
        agg = a0_ref[...] + a1_ref[...]
        o_ref[...] = _elu(
            jnp.dot(h_ref[...], wr_ref[...], preferred_element_type=jnp.float32)
            + jnp.dot(agg, wl_ref[...], preferred_element_type=jnp.float32)
            + b_ref[...])

    return pl.pallas_call(
        body,
        grid=(NPAD // BM,),
        in_specs=[
            pl.BlockSpec((BM, W), lambda i: (i, 0)),
            pl.BlockSpec((BM, W), lambda i: (i, 0)),
            pl.BlockSpec((BM, W), lambda i: (i, 0)),
            pl.BlockSpec((W, W), lambda i: (0, 0)),
            pl.BlockSpec((W, W), lambda i: (0, 0)),
            pl.BlockSpec((1, W), lambda i: (0, 0)),
        ],
        out_specs=pl.BlockSpec((BM, W), lambda i: (i, 0)),
        out_shape=jax.ShapeDtypeStruct((NPAD, W), jnp.float32),
    )(h, agg0, agg1, w_root, w_rel, b.reshape(1, W))


def _head(p0, p1, fc1_w, fc1_b, fc2_w, fc2_b, fc3_w, fc3_b):
    SB = 1000

    def body(p0_ref, p1_ref, w1_ref, b1_ref, w2_ref, b2_ref, w3_ref, b3_ref, o_ref):
        p = p0_ref[...] + p1_ref[...]
        h = _elu(jnp.dot(p, w1_ref[...], preferred_element_type=jnp.float32) + b1_ref[...])
        h = _elu(jnp.dot(h, w2_ref[...], preferred_element_type=jnp.float32) + b2_ref[...])
        lg = jnp.dot(h, w3_ref[...], preferred_element_type=jnp.float32) + b3_ref[...]
        m = jnp.max(lg, axis=1, keepdims=True)
        e = jnp.exp(lg - m)
        o_ref[...] = (lg - m) - jnp.log(jnp.sum(e, axis=1, keepdims=True))

    return pl.pallas_call(
        body,
        grid=(S // SB,),
        in_specs=[
            pl.BlockSpec((SB, W), lambda i: (i, 0)),
            pl.BlockSpec((SB, W), lambda i: (i, 0)),
            pl.BlockSpec((W, W), lambda i: (0, 0)),
            pl.BlockSpec((1, W), lambda i: (0, 0)),
            pl.BlockSpec((W, 32), lambda i: (0, 0)),
            pl.BlockSpec((1, 32), lambda i: (0, 0)),
            pl.BlockSpec((32, 2), lambda i: (0, 0)),
            pl.BlockSpec((1, 2), lambda i: (0, 0)),
        ],
        out_specs=pl.BlockSpec((SB, 2), lambda i: (i, 0)),
        out_shape=jax.ShapeDtypeStruct((S, 2), jnp.float32),
    )(p0, p1, fc1_w, fc1_b.reshape(1, W), fc2_w, fc2_b.reshape(1, 32),
      fc3_w, fc3_b.reshape(1, 2))


def kernel(x, edge_index, assign_nodes, assign_set_ids,
           W0_root, W0_rel, b0,
           W1_root, W1_rel, b1,
           W2_root, W2_rel, b2,
           fc1_w, fc1_b, fc2_w, fc2_b, fc3_w, fc3_b):
    src = edge_index[0].astype(jnp.int32)
    dst = edge_index[1].astype(jnp.int32)
    eidx = _pack_idx(src, dst, NC * NT * ECHUNKS_PER_TILE * CH, NPAD - 1)
    aidx = _pack_idx(assign_set_ids, assign_nodes,
                     NC * NT * ACHUNKS_PER_TILE * CH, SPAD - 1)

    xp = jnp.concatenate([x, jnp.zeros((NPAD - N, D), jnp.float32)])
    x2 = jnp.concatenate([xp[:, :W], xp[:, W:]])
    a0 = _seg_l0(x2, eidx)
    h1 = _layer0_tc(xp, a0[:NPAD], a0[NPAD:], W0_root, W0_rel, b0)
    agg = _seg_edge(h1, eidx)
    h2 = _layer_tc(h1, agg[:NPAD], agg[NPAD:], W1_root, W1_rel, b1)
    agg = _seg_edge(h2, eidx)
    h3 = _layer_tc(h2, agg[:NPAD], agg[NPAD:], W2_root, W2_rel, b2)

    pooled = _seg_pool(h3, aidx)
    return _head(pooled[:S], pooled[SPAD:SPAD + S],
                 fc1_w, fc1_b, fc2_w, fc2_b, fc3_w, fc3_b)


# speedup vs baseline: 2.6410x; 1.0472x over previous
import functools

import jax
import jax.numpy as jnp
from jax import lax
from jax.experimental import pallas as pl
from jax.experimental.pallas import tpu as pltpu
from jax.experimental.pallas import tpu_sc as plsc

N = 10000
E = 320000
D = 128
W = 64
A = 40000
S = 5000

NC = 2
NT = 16
CH = 128

NPAD = 10112
SPAD = 5120

ECHUNKS_PER_TILE = 80
ACHUNKS_PER_TILE = 10

BM = 632


def _elu(v):
    return jnp.where(v > 0, v, jnp.exp(jnp.minimum(v, 0.0)) - 1.0)


def _make_sc_segment_sum(npad, chunks_per_tile):
    mesh = plsc.VectorSubcoreMesh(core_axis_name="c", subcore_axis_name="s")
    zr = npad // NT
    tr = NPAD // NT
    nb = chunks_per_tile // 2

    @functools.partial(
        pl.kernel,
        out_type=jax.ShapeDtypeStruct((NC * npad, W), jnp.float32),
        mesh=mesh,
        scratch_types=[
            pltpu.VMEM((chunks_per_tile, 2, CH), jnp.int32),
            pltpu.VMEM((CH, W), jnp.float32),
            pltpu.VMEM((CH, W), jnp.float32),
            pltpu.SemaphoreType.DMA,
            pltpu.SemaphoreType.DMA,
            pltpu.SemaphoreType.DMA,
            pltpu.SemaphoreType.DMA,
            pltpu.SemaphoreType.DMA,
            pltpu.SemaphoreType.DMA,
            pltpu.VMEM_SHARED((NPAD, W), jnp.float32),
            pltpu.VMEM_SHARED((npad, W), jnp.float32),
        ],
        compiler_params=pltpu.CompilerParams(use_tc_tiling_on_sc=False),
    )
    def seg(table_hbm, idx_hbm, out_hbm, idxall, rows0, rows1,
            sem_i, sem_t, sem_g0, sem_g1, sem_s0, sem_s1, tbl, acc):
        cid = lax.axis_index("c")
        sid = lax.axis_index("s")
        c0 = (cid * NT + sid) * chunks_per_tile

        idx_cp = pltpu.async_copy(idx_hbm.at[pl.ds(c0, chunks_per_tile)],
                                  idxall, sem_i)
        tbase = pl.multiple_of(sid * tr, 8)
        tbl_cp = pltpu.async_copy(table_hbm.at[pl.ds(tbase, tr)],
                                  tbl.at[pl.ds(tbase, tr)], sem_t)
        z16 = jnp.zeros((16,), jnp.float32)

        def zbody(i, carry):
            for k in range(W // 16):
                rows0[i, pl.ds(k * 16, 16)] = z16
            return carry

        lax.fori_loop(0, CH, zbody, 0)
        zbase = pl.multiple_of(sid * zr, 8)
        nfull, rem = zr // CH, zr % CH
        for q in range(nfull):
            pltpu.sync_copy(rows0, acc.at[pl.ds(zbase + q * CH, CH)])
        if rem:
            pltpu.sync_copy(rows0.at[pl.ds(0, rem)],
                            acc.at[pl.ds(zbase + nfull * CH, rem)])
        idx_cp.wait()
        tbl_cp.wait()
        plsc.subcore_barrier()
        pltpu.async_copy(tbl.at[idxall.at[0, 0]], rows0, sem_g0)

        def body(k, carry):
            j0 = 2 * k
            j1 = j0 + 1
            pltpu.make_async_copy(tbl.at[idxall.at[j0, 0]],
                                  rows0, sem_g0).wait()
            pltpu.async_copy(rows0, acc.at[idxall.at[j0, 1]], sem_s0, add=True)

            @pl.when(k > 0)
            def _():
                pltpu.make_async_copy(rows1, acc.at[idxall.at[j1, 1]],
                                      sem_s1).wait()

            pltpu.async_copy(tbl.at[idxall.at[j1, 0]], rows1, sem_g1)
            pltpu.make_async_copy(tbl.at[idxall.at[j1, 0]],
                                  rows1, sem_g1).wait()
            pltpu.async_copy(rows1, acc.at[idxall.at[j1, 1]], sem_s1, add=True)
            pltpu.make_async_copy(rows0, acc.at[idxall.at[j0, 1]],
                                  sem_s0).wait()

            @pl.when(k < nb - 1)
            def _():
                pltpu.async_copy(tbl.at[idxall.at[j0 + 2, 0]],
                                 rows0, sem_g0)

            return carry

        lax.fori_loop(0, nb, body, 0)
        pltpu.make_async_copy(rows1, acc.at[idxall.at[0, 1]], sem_s1).wait()
        plsc.subcore_barrier()

        pltpu.sync_copy(acc.at[pl.ds(zbase, zr)],
                        out_hbm.at[pl.ds(pl.multiple_of(cid * npad + zbase, 8), zr)])

    return seg


_seg_edge = _make_sc_segment_sum(NPAD, ECHUNKS_PER_TILE)
_seg_pool = _make_sc_segment_sum(SPAD, ACHUNKS_PER_TILE)


def _make_sc_l0():
    mesh = plsc.VectorSubcoreMesh(core_axis_name="c", subcore_axis_name="s")
    cpt = NC * ECHUNKS_PER_TILE
    hcpt = cpt // 2
    zr = NPAD // NT
    nb = hcpt // 2

    @functools.partial(
        pl.kernel,
        out_type=jax.ShapeDtypeStruct((NC * NPAD, W), jnp.float32),
        mesh=mesh,
        scratch_types=[
            pltpu.VMEM((hcpt, 2, CH), jnp.int32),
            pltpu.VMEM((CH, W), jnp.float32),
            pltpu.VMEM((CH, W), jnp.float32),
            pltpu.SemaphoreType.DMA,
            pltpu.SemaphoreType.DMA,
            pltpu.SemaphoreType.DMA,
            pltpu.SemaphoreType.DMA,
            pltpu.SemaphoreType.DMA,
            pltpu.SemaphoreType.DMA,
            pltpu.VMEM_SHARED((NPAD, W), jnp.float32),
            pltpu.VMEM_SHARED((NPAD, W), jnp.float32),
        ],
        compiler_params=pltpu.CompilerParams(use_tc_tiling_on_sc=False),
    )
    def seg(x_hbm, idx_hbm, out_hbm, idxall, rows0, rows1,
            sem_i, sem_t, sem_g0, sem_g1, sem_s0, sem_s1, tbl, acc):
        cid = lax.axis_index("c")
        sid = lax.axis_index("s")
        c0 = sid * cpt

        idx_cp = pltpu.async_copy(idx_hbm.at[pl.ds(c0, hcpt)], idxall, sem_i)
        tbase = pl.multiple_of(sid * zr, 8)
        tbl_cp = pltpu.async_copy(
            x_hbm.at[pl.ds(tbase, zr), pl.ds(pl.multiple_of(cid * W, W), W)],
            tbl.at[pl.ds(tbase, zr)], sem_t)
        z16 = jnp.zeros((16,), jnp.float32)

        def zbody(i, carry):
            for k in range(W // 16):
                rows0[i, pl.ds(k * 16, 16)] = z16
            return carry

        lax.fori_loop(0, CH, zbody, 0)
        zbase = pl.multiple_of(sid * zr, 8)
        nfull, rem = zr // CH, zr % CH
        for q in range(nfull):
            pltpu.sync_copy(rows0, acc.at[pl.ds(zbase + q * CH, CH)])
        if rem:
            pltpu.sync_copy(rows0.at[pl.ds(0, rem)],
                            acc.at[pl.ds(zbase + nfull * CH, rem)])
        idx_cp.wait()
        tbl_cp.wait()
        plsc.subcore_barrier()

        def body(k, carry):
            j0 = 2 * k
            j1 = j0 + 1
            pltpu.make_async_copy(tbl.at[idxall.at[j0, 0]],
                                  rows0, sem_g0).wait()
            pltpu.async_copy(rows0, acc.at[idxall.at[j0, 1]], sem_s0, add=True)

            @pl.when(k > 0)
            def _():
                pltpu.make_async_copy(rows1, acc.at[idxall.at[j1, 1]],
                                      sem_s1).wait()

            pltpu.async_copy(tbl.at[idxall.at[j1, 0]], rows1, sem_g1)
            pltpu.make_async_copy(tbl.at[idxall.at[j1, 0]],
                                  rows1, sem_g1).wait()
            pltpu.async_copy(rows1, acc.at[idxall.at[j1, 1]], sem_s1, add=True)
            pltpu.make_async_copy(rows0, acc.at[idxall.at[j0, 1]],
                                  sem_s0).wait()

            @pl.when(k < nb - 1)
            def _():
                pltpu.async_copy(tbl.at[idxall.at[j0 + 2, 0]],
                                 rows0, sem_g0)

            return carry

        pltpu.async_copy(tbl.at[idxall.at[0, 0]], rows0, sem_g0)
        lax.fori_loop(0, nb, body, 0)
        pltpu.make_async_copy(rows1, acc.at[idxall.at[0, 1]], sem_s1).wait()

        pltpu.sync_copy(idx_hbm.at[pl.ds(c0 + hcpt, hcpt)], idxall)
        pltpu.async_copy(tbl.at[idxall.at[0, 0]], rows0, sem_g0)
        lax.fori_loop(0, nb, body, 0)
        pltpu.make_async_copy(rows1, acc.at[idxall.at[0, 1]], sem_s1).wait()

        plsc.subcore_barrier()
        pltpu.sync_copy(acc.at[pl.ds(zbase, zr)],
                        out_hbm.at[pl.ds(pl.multiple_of(cid * NPAD + zbase, 8), zr)])

    return seg


_seg_l0 = _make_sc_l0()


def _pack_idx(gidx, sidx, total, dump_row):
    n = gidx.shape[0]
    pad = total - n
    g = jnp.concatenate([gidx.astype(jnp.int32),
                         jnp.zeros((pad,), jnp.int32)])
    s = jnp.concatenate([sidx.astype(jnp.int32),
                         jnp.full((pad,), dump_row, jnp.int32)])
    return jnp.stack([g.reshape(-1, CH), s.reshape(-1, CH)], axis=1)


def _layer0_tc(x, alo, ahi, w_root, w_rel, b):
    def body(x_ref, lo, hi, wr_ref, wl_ref, b_ref, o_ref):
        agg = jnp.concatenate([lo[...], hi[...]], axis=1)
        o_ref[...] = _elu(
            jnp.dot(x_ref[...], wr_ref[...], preferred_element_type=jnp.float32)
            + jnp.dot(agg, wl_ref[...], preferred_element_type=jnp.float32)
            + b_ref[...])

    return pl.pallas_call(
        body,
        grid=(NPAD // BM,),
        in_specs=[
            pl.BlockSpec((BM, D), lambda i: (i, 0)),
            pl.BlockSpec((BM, W), lambda i: (i, 0)),
            pl.BlockSpec((BM, W), lambda i: (i, 0)),
            pl.BlockSpec((D, W), lambda i: (0, 0)),
            pl.BlockSpec((D, W), lambda i: (0, 0)),
            pl.BlockSpec((1, W), lambda i: (0, 0)),
        ],
        out_specs=pl.BlockSpec((BM, W), lambda i: (i, 0)),
        out_shape=jax.ShapeDtypeStruct((NPAD, W), jnp.float32),
    )(x, alo, ahi, w_root, w_rel, b.reshape(1, W))


def _layer_tc(h, agg0, agg1, w_root, w_rel, b):
    def body(h_ref, a0_ref, a1_ref, wr_ref, wl_ref, b_ref, o_ref):
        agg = a0_ref[...] + a1_ref[...]
        o_ref[...] = _elu(
            jnp.dot(h_ref[...], wr_ref[...], preferred_element_type=jnp.float32)
            + jnp.dot(agg, wl_ref[...], preferred_element_type=jnp.float32)
            + b_ref[...])

    return pl.pallas_call(
        body,
        grid=(NPAD // BM,),
        in_specs=[
            pl.BlockSpec((BM, W), lambda i: (i, 0)),
            pl.BlockSpec((BM, W), lambda i: (i, 0)),
            pl.BlockSpec((BM, W), lambda i: (i, 0)),
            pl.BlockSpec((W, W), lambda i: (0, 0)),
            pl.BlockSpec((W, W), lambda i: (0, 0)),
            pl.BlockSpec((1, W), lambda i: (0, 0)),
        ],
        out_specs=pl.BlockSpec((BM, W), lambda i: (i, 0)),
        out_shape=jax.ShapeDtypeStruct((NPAD, W), jnp.float32),
    )(h, agg0, agg1, w_root, w_rel, b.reshape(1, W))


def _head(p0, p1, fc1_w, fc1_b, fc2_w, fc2_b, fc3_w, fc3_b):
    SB = 1000

    def body(p0_ref, p1_ref, w1_ref, b1_ref, w2_ref, b2_ref, w3_ref, b3_ref, o_ref):
        p = p0_ref[...] + p1_ref[...]
        h = _elu(jnp.dot(p, w1_ref[...], preferred_element_type=jnp.float32) + b1_ref[...])
        h = _elu(jnp.dot(h, w2_ref[...], preferred_element_type=jnp.float32) + b2_ref[...])
        lg = jnp.dot(h, w3_ref[...], preferred_element_type=jnp.float32) + b3_ref[...]
        m = jnp.max(lg, axis=1, keepdims=True)
        e = jnp.exp(lg - m)
        o_ref[...] = (lg - m) - jnp.log(jnp.sum(e, axis=1, keepdims=True))

    return pl.pallas_call(
        body,
        grid=(S // SB,),
        in_specs=[
            pl.BlockSpec((SB, W), lambda i: (i, 0)),
            pl.BlockSpec((SB, W), lambda i: (i, 0)),
            pl.BlockSpec((W, W), lambda i: (0, 0)),
            pl.BlockSpec((1, W), lambda i: (0, 0)),
            pl.BlockSpec((W, 32), lambda i: (0, 0)),
            pl.BlockSpec((1, 32), lambda i: (0, 0)),
            pl.BlockSpec((32, 2), lambda i: (0, 0)),
            pl.BlockSpec((1, 2), lambda i: (0, 0)),
        ],
        out_specs=pl.BlockSpec((SB, 2), lambda i: (i, 0)),
        out_shape=jax.ShapeDtypeStruct((S, 2), jnp.float32),
    )(p0, p1, fc1_w, fc1_b.reshape(1, W), fc2_w, fc2_b.reshape(1, 32),
      fc3_w, fc3_b.reshape(1, 2))


def kernel(x, edge_index, assign_nodes, assign_set_ids,
           W0_root, W0_rel, b0,
           W1_root, W1_rel, b1,
           W2_root, W2_rel, b2,
           fc1_w, fc1_b, fc2_w, fc2_b, fc3_w, fc3_b):
    src = edge_index[0].astype(jnp.int32)
    dst = edge_index[1].astype(jnp.int32)
    eidx = _pack_idx(src, dst, NC * NT * ECHUNKS_PER_TILE * CH, NPAD - 1)
    aidx = _pack_idx(assign_set_ids, assign_nodes,
                     NC * NT * ACHUNKS_PER_TILE * CH, SPAD - 1)

    xp = jnp.concatenate([x, jnp.zeros((NPAD - N, D), jnp.float32)])
    a0 = _seg_l0(xp, eidx)
    h1 = _layer0_tc(xp, a0[:NPAD], a0[NPAD:], W0_root, W0_rel, b0)
    agg = _seg_edge(h1, eidx)
    h2 = _layer_tc(h1, agg[:NPAD], agg[NPAD:], W1_root, W1_rel, b1)
    agg = _seg_edge(h2, eidx)
    h3 = _layer_tc(h2, agg[:NPAD], agg[NPAD:], W2_root, W2_rel, b2)

    pooled = _seg_pool(h3, aidx)
    return _head(pooled[:S], pooled[SPAD:SPAD + S],
                 fc1_w, fc1_b, fc2_w, fc2_b, fc3_w, fc3_b)
